# Initial kernel scaffold; baseline (speedup 1.0000x reference)
import jax
import jax.numpy as jnp
from jax.experimental import pallas as pl


def kernel(x):
    raise NotImplementedError("write your pallas kernel here")


# speedup vs baseline: 2.5190x; 2.5190x over previous
import functools

import numpy as np
import jax
import jax.numpy as jnp
from jax import lax
from jax.experimental import pallas as pl
from jax.experimental.pallas import tpu as pltpu
from jax.experimental.pallas import tpu_sc as plsc

_B, _N, _D = 64, 576, 768
_NKEEP = 144
_MASK_RATIO_CHECK = _NKEEP == int(_N * (1.0 - 0.75))

_NC, _NS = 2, 16
_NW = _NC * _NS
_ROWS = _B * _NKEEP
_RPW = _ROWS // _NW
_CHUNK = 72
_NCHUNK = _RPW // _CHUNK

_consts_cache = None


def _threefry_noise(shape):
    n = int(np.prod(shape))
    with np.errstate(over="ignore"):
        x = [np.zeros(n, np.uint32), np.arange(n, dtype=np.uint32)]
        k0, k1 = np.uint32(0), np.uint32(42)
        rotations = [(13, 15, 26, 6), (17, 29, 16, 24)]
        ks = [k0, k1, np.uint32(k0 ^ k1 ^ np.uint32(0x1BD11BDA))]

        def rotl(v, d):
            return (v << np.uint32(d)) | (v >> np.uint32(32 - d))

        x[0] = x[0] + ks[0]
        x[1] = x[1] + ks[1]
        for i in range(5):
            for r in rotations[i % 2]:
                x[0] = x[0] + x[1]
                x[1] = rotl(x[1], r)
                x[1] = x[1] ^ x[0]
            x[0] = x[0] + ks[(i + 1) % 3]
            x[1] = x[1] + ks[(i + 2) % 3] + np.uint32(i + 1)
        bits = x[0] ^ x[1]
    f = ((bits >> np.uint32(9)) | np.uint32(0x3F800000)).view(np.float32)
    f = f - np.float32(1.0)
    return np.maximum(np.float32(0.0), f).reshape(shape)


def _constants():
    global _consts_cache
    if _consts_cache is None:
        noise = _threefry_noise((_B, _N))
        sorted_idx = np.argsort(noise, axis=1, kind="stable").astype(np.int32)
        pos_idx = np.argsort(sorted_idx, axis=1, kind="stable").astype(np.int32)
        mask = (pos_idx >= _NKEEP).astype(np.float32)
        keep = sorted_idx[:, :_NKEEP].astype(np.int32)
        flat_idx = keep + np.arange(_B, dtype=np.int32)[:, None] * _N
        idx = flat_idx.reshape(_NW, _NCHUNK, _CHUNK)
        _consts_cache = (idx, mask, pos_idx, sorted_idx)
    return _consts_cache


def _gather(x_flat, idx):
    mesh = plsc.VectorSubcoreMesh(core_axis_name="c", subcore_axis_name="s")

    @functools.partial(
        pl.kernel,
        mesh=mesh,
        out_type=jax.ShapeDtypeStruct((_ROWS, _D), jnp.float32),
        scratch_types=[
            pltpu.VMEM((_NCHUNK, _CHUNK), jnp.int32),
            pltpu.VMEM((_CHUNK, _D), jnp.float32),
        ],
    )
    def k(x_hbm, idx_hbm, out_hbm, idx_v, buf):
        wid = lax.axis_index("s") * _NC + lax.axis_index("c")
        base = wid * _RPW
        pltpu.sync_copy(idx_hbm.at[wid], idx_v)
        for c in range(_NCHUNK):
            pltpu.sync_copy(x_hbm.at[idx_v.at[c]], buf)
            pltpu.sync_copy(buf, out_hbm.at[pl.ds(base + c * _CHUNK, _CHUNK)])

    return k(x_flat, idx)


def kernel(x):
    idx, mask, pos_idx, sorted_idx = _constants()
    x_flat = x.reshape(_B * _N, _D)
    out = _gather(x_flat, jnp.asarray(idx))
    x_keep = out.reshape(_B, _NKEEP, _D)
    return (
        x_keep,
        jnp.asarray(mask),
        jnp.asarray(pos_idx),
        jnp.asarray(sorted_idx),
    )


# speedup vs baseline: 2.6285x; 1.0434x over previous
import functools

import numpy as np
import jax
import jax.numpy as jnp
from jax import lax
from jax.experimental import pallas as pl
from jax.experimental.pallas import tpu as pltpu
from jax.experimental.pallas import tpu_sc as plsc

_B, _N, _D = 64, 576, 768
_NKEEP = 144
_MASK_RATIO_CHECK = _NKEEP == int(_N * (1.0 - 0.75))

_NC, _NS = 2, 16
_NW = _NC * _NS
_ROWS = _B * _NKEEP
_RPW = _ROWS // _NW
_CHUNK = 72
_NCHUNK = _RPW // _CHUNK

_consts_cache = None


def _threefry_noise(shape):
    n = int(np.prod(shape))
    with np.errstate(over="ignore"):
        x = [np.zeros(n, np.uint32), np.arange(n, dtype=np.uint32)]
        k0, k1 = np.uint32(0), np.uint32(42)
        rotations = [(13, 15, 26, 6), (17, 29, 16, 24)]
        ks = [k0, k1, np.uint32(k0 ^ k1 ^ np.uint32(0x1BD11BDA))]

        def rotl(v, d):
            return (v << np.uint32(d)) | (v >> np.uint32(32 - d))

        x[0] = x[0] + ks[0]
        x[1] = x[1] + ks[1]
        for i in range(5):
            for r in rotations[i % 2]:
                x[0] = x[0] + x[1]
                x[1] = rotl(x[1], r)
                x[1] = x[1] ^ x[0]
            x[0] = x[0] + ks[(i + 1) % 3]
            x[1] = x[1] + ks[(i + 2) % 3] + np.uint32(i + 1)
        bits = x[0] ^ x[1]
    f = ((bits >> np.uint32(9)) | np.uint32(0x3F800000)).view(np.float32)
    f = f - np.float32(1.0)
    return np.maximum(np.float32(0.0), f).reshape(shape)


def _constants():
    global _consts_cache
    if _consts_cache is None:
        noise = _threefry_noise((_B, _N))
        sorted_idx = np.argsort(noise, axis=1, kind="stable").astype(np.int32)
        pos_idx = np.argsort(sorted_idx, axis=1, kind="stable").astype(np.int32)
        mask = (pos_idx >= _NKEEP).astype(np.float32)
        keep = sorted_idx[:, :_NKEEP].astype(np.int32)
        flat_idx = keep + np.arange(_B, dtype=np.int32)[:, None] * _N
        idx = flat_idx.reshape(_NW, _NCHUNK, _CHUNK)
        _consts_cache = (idx, mask, pos_idx, sorted_idx)
    return _consts_cache


def _gather(x_flat, idx):
    mesh = plsc.VectorSubcoreMesh(core_axis_name="c", subcore_axis_name="s")

    @functools.partial(
        pl.kernel,
        mesh=mesh,
        out_type=jax.ShapeDtypeStruct((_ROWS, _D), jnp.float32),
        scratch_types=[
            pltpu.VMEM((_NCHUNK, _CHUNK), jnp.int32),
            pltpu.VMEM((_CHUNK, _D), jnp.float32),
            pltpu.VMEM((_CHUNK, _D), jnp.float32),
            pltpu.SemaphoreType.DMA,
            pltpu.SemaphoreType.DMA,
            pltpu.SemaphoreType.DMA,
            pltpu.SemaphoreType.DMA,
        ],
    )
    def k(x_hbm, idx_hbm, out_hbm, idx_v, buf0, buf1, g0, g1, s0, s1):
        wid = lax.axis_index("s") * _NC + lax.axis_index("c")
        base = wid * _RPW
        bufs, gsem, ssem = [buf0, buf1], [g0, g1], [s0, s1]
        pltpu.sync_copy(idx_hbm.at[wid], idx_v)
        gathers = [None, None]
        scatters = [None, None]
        gathers[0] = pltpu.async_copy(x_hbm.at[idx_v.at[0]], bufs[0], gsem[0])
        for c in range(_NCHUNK):
            cur, nxt = c % 2, (c + 1) % 2
            if c + 1 < _NCHUNK:
                if scatters[nxt] is not None:
                    scatters[nxt].wait()
                gathers[nxt] = pltpu.async_copy(
                    x_hbm.at[idx_v.at[c + 1]], bufs[nxt], gsem[nxt]
                )
            gathers[cur].wait()
            scatters[cur] = pltpu.async_copy(
                bufs[cur], out_hbm.at[pl.ds(base + c * _CHUNK, _CHUNK)], ssem[cur]
            )
        scatters[0].wait()
        scatters[1].wait()

    return k(x_flat, idx)


def kernel(x):
    idx, mask, pos_idx, sorted_idx = _constants()
    x_flat = x.reshape(_B * _N, _D)
    out = _gather(x_flat, jnp.asarray(idx))
    x_keep = out.reshape(_B, _NKEEP, _D)
    return (
        x_keep,
        jnp.asarray(mask),
        jnp.asarray(pos_idx),
        jnp.asarray(sorted_idx),
    )


# speedup vs baseline: 2.6717x; 1.0165x over previous
import functools

import numpy as np
import jax
import jax.numpy as jnp
from jax import lax
from jax.experimental import pallas as pl
from jax.experimental.pallas import tpu as pltpu
from jax.experimental.pallas import tpu_sc as plsc

_B, _N, _D = 64, 576, 768
_NKEEP = 144
_MASK_RATIO_CHECK = _NKEEP == int(_N * (1.0 - 0.75))

_NC, _NS = 2, 16
_NW = _NC * _NS
_ROWS = _B * _NKEEP
_RPW = _ROWS // _NW
_CHUNK = 48
_NCHUNK = _RPW // _CHUNK
_NBUF = 3

_consts_cache = None


def _threefry_noise(shape):
    n = int(np.prod(shape))
    with np.errstate(over="ignore"):
        x = [np.zeros(n, np.uint32), np.arange(n, dtype=np.uint32)]
        k0, k1 = np.uint32(0), np.uint32(42)
        rotations = [(13, 15, 26, 6), (17, 29, 16, 24)]
        ks = [k0, k1, np.uint32(k0 ^ k1 ^ np.uint32(0x1BD11BDA))]

        def rotl(v, d):
            return (v << np.uint32(d)) | (v >> np.uint32(32 - d))

        x[0] = x[0] + ks[0]
        x[1] = x[1] + ks[1]
        for i in range(5):
            for r in rotations[i % 2]:
                x[0] = x[0] + x[1]
                x[1] = rotl(x[1], r)
                x[1] = x[1] ^ x[0]
            x[0] = x[0] + ks[(i + 1) % 3]
            x[1] = x[1] + ks[(i + 2) % 3] + np.uint32(i + 1)
        bits = x[0] ^ x[1]
    f = ((bits >> np.uint32(9)) | np.uint32(0x3F800000)).view(np.float32)
    f = f - np.float32(1.0)
    return np.maximum(np.float32(0.0), f).reshape(shape)


def _constants():
    global _consts_cache
    if _consts_cache is None:
        noise = _threefry_noise((_B, _N))
        sorted_idx = np.argsort(noise, axis=1, kind="stable").astype(np.int32)
        pos_idx = np.argsort(sorted_idx, axis=1, kind="stable").astype(np.int32)
        mask = (pos_idx >= _NKEEP).astype(np.float32)
        keep = sorted_idx[:, :_NKEEP].astype(np.int32)
        flat_idx = keep + np.arange(_B, dtype=np.int32)[:, None] * _N
        idx = flat_idx.reshape(_NW, _NCHUNK, _CHUNK)
        _consts_cache = (idx, mask, pos_idx, sorted_idx)
    return _consts_cache


def _gather(x_flat, idx):
    mesh = plsc.VectorSubcoreMesh(core_axis_name="c", subcore_axis_name="s")

    @functools.partial(
        pl.kernel,
        mesh=mesh,
        out_type=jax.ShapeDtypeStruct((_ROWS, _D), jnp.float32),
        scratch_types=(
            [pltpu.VMEM((_NCHUNK, _CHUNK), jnp.int32)]
            + [pltpu.VMEM((_CHUNK, _D), jnp.float32)] * _NBUF
            + [pltpu.SemaphoreType.DMA] * (2 * _NBUF)
        ),
    )
    def k(x_hbm, idx_hbm, out_hbm, idx_v, *scratch):
        bufs = list(scratch[:_NBUF])
        gsem = list(scratch[_NBUF:2 * _NBUF])
        ssem = list(scratch[2 * _NBUF:])
        wid = lax.axis_index("s") * _NC + lax.axis_index("c")
        base = wid * _RPW
        pltpu.sync_copy(idx_hbm.at[wid], idx_v)
        gathers = [None] * _NBUF
        scatters = [None] * _NBUF
        for c in range(min(_NBUF, _NCHUNK)):
            gathers[c] = pltpu.async_copy(
                x_hbm.at[idx_v.at[c]], bufs[c], gsem[c]
            )
        for c in range(_NCHUNK):
            cur = c % _NBUF
            gathers[cur].wait()
            scatters[cur] = pltpu.async_copy(
                bufs[cur], out_hbm.at[pl.ds(base + c * _CHUNK, _CHUNK)], ssem[cur]
            )
            nc = c + _NBUF
            if nc < _NCHUNK:
                scatters[cur].wait()
                gathers[cur] = pltpu.async_copy(
                    x_hbm.at[idx_v.at[nc]], bufs[cur], gsem[cur]
                )
        for c in range(max(0, _NCHUNK - _NBUF), _NCHUNK):
            scatters[c % _NBUF].wait()

    return k(x_flat, idx)


def kernel(x):
    idx, mask, pos_idx, sorted_idx = _constants()
    x_flat = x.reshape(_B * _N, _D)
    out = _gather(x_flat, jnp.asarray(idx))
    x_keep = out.reshape(_B, _NKEEP, _D)
    return (
        x_keep,
        jnp.asarray(mask),
        jnp.asarray(pos_idx),
        jnp.asarray(sorted_idx),
    )
